# baseline (speedup 1.0000x reference)
import functools

import jax
import jax.numpy as jnp
from jax import lax
from jax.experimental import pallas as pl
from jax.experimental.pallas import tpu as pltpu
from jax.experimental.pallas import tpu_sc as plsc

N_TOK = 8192
D_IN = 4096
D_OUT = 4096
N_EXP = 8
N_ASSIGN = 2 * N_TOK

M_TILE = 256
M_TILES = N_ASSIGN // M_TILE
N_WORKS = M_TILES + N_EXP - 1
N_TILE = 2048
N_TILES = D_OUT // N_TILE

SC_CORES = 2
SC_SUBCORES = 16
SC_WORKERS = SC_CORES * SC_SUBCORES

GATE_BM = 512


def _gating_body(x_ref, wg_ref, bg_ref, i0_ref, i1_ref, w0_ref, w1_ref):
    g = jax.lax.dot_general(
        x_ref[...], wg_ref[...], (((1,), (0,)), ((), ())),
        preferred_element_type=jnp.float32)
    g = g + bg_ref[...]
    lanes = jax.lax.broadcasted_iota(jnp.int32, (GATE_BM, 128), 1)
    real = lanes < N_EXP
    neg = jnp.where(real, g, -jnp.inf)
    m0 = jnp.max(neg, axis=1, keepdims=True)
    i0 = jnp.max(jnp.where((neg == m0) & real, lanes, -1), axis=1, keepdims=True)
    neg2 = jnp.where(lanes == i0, -jnp.inf, neg)
    m1 = jnp.max(neg2, axis=1, keepdims=True)
    i1 = jnp.max(jnp.where((neg2 == m1) & real, lanes, -1), axis=1, keepdims=True)
    e1 = jnp.exp(m1 - m0)
    denom = 1.0 + e1
    i0_ref[...] = i0
    i1_ref[...] = i1
    w0_ref[...] = 1.0 / denom
    w1_ref[...] = e1 / denom


def _gating(x, wg_pad, bg_pad):
    grid = (N_TOK // GATE_BM,)
    out1 = jax.ShapeDtypeStruct((N_TOK, 1), jnp.int32)
    outf = jax.ShapeDtypeStruct((N_TOK, 1), jnp.float32)
    return pl.pallas_call(
        _gating_body,
        grid=grid,
        in_specs=[
            pl.BlockSpec((GATE_BM, D_IN), lambda m: (m, 0)),
            pl.BlockSpec((D_IN, 128), lambda m: (0, 0)),
            pl.BlockSpec((1, 128), lambda m: (0, 0)),
        ],
        out_specs=[
            pl.BlockSpec((GATE_BM, 1), lambda m: (m, 0)),
            pl.BlockSpec((GATE_BM, 1), lambda m: (m, 0)),
            pl.BlockSpec((GATE_BM, 1), lambda m: (m, 0)),
            pl.BlockSpec((GATE_BM, 1), lambda m: (m, 0)),
        ],
        out_shape=[out1, out1, outf, outf],
    )(x, wg_pad, bg_pad)


def _routing_tables(i0, i1, w0, w1):
    e_flat = jnp.stack([i0, i1], axis=1).reshape(-1).astype(jnp.int32)
    onehot = (e_flat[:, None] == jnp.arange(N_EXP, dtype=jnp.int32)[None, :])
    ends = jnp.cumsum(onehot.astype(jnp.int32), axis=0)
    counts = ends[-1]
    offs = jnp.concatenate(
        [jnp.zeros((1,), jnp.int32), jnp.cumsum(counts)]).astype(jnp.int32)
    rank = jnp.take_along_axis(ends, e_flat[:, None], axis=1)[:, 0] - 1
    pos = (offs[e_flat] + rank).astype(jnp.int32)
    p0 = pos[0::2]
    p1 = pos[1::2]
    w_flat = jnp.stack([w0, w1], axis=1).reshape(-1)
    wbits = lax.bitcast_convert_type(w_flat, jnp.int32)
    packed = jnp.zeros((N_ASSIGN, 2), jnp.int32).at[pos].set(
        jnp.stack([jnp.arange(N_ASSIGN, dtype=jnp.int32) // 2, wbits],
                  axis=1))
    tok_sorted = packed[:, 0]
    ws_sorted = lax.bitcast_convert_type(packed[:, 1:2], jnp.float32)

    first_tile = offs[:N_EXP] // M_TILE
    last_tile = (offs[1:] - 1) // M_TILE
    ntiles = jnp.where(counts > 0, last_tile - first_tile + 1, 0)
    wend = jnp.cumsum(ntiles)
    wstart = jnp.concatenate([jnp.zeros((1,), wend.dtype), wend[:-1]])
    total = wend[N_EXP - 1]
    w_ar = jnp.arange(N_WORKS, dtype=jnp.int32)
    ew = jnp.minimum(jnp.searchsorted(wend, w_ar, side="right"), N_EXP - 1)
    mw = first_tile[ew] + (w_ar - wstart[ew])
    el = jnp.minimum(jnp.searchsorted(wend, total - 1, side="right"), N_EXP - 1)
    ml = first_tile[el] + (total - 1 - wstart[el])
    valid = w_ar < total
    e_ids = jnp.where(valid, ew, el).astype(jnp.int32)
    m_ids = jnp.where(valid, mw, ml).astype(jnp.int32)
    return tok_sorted, p0, p1, ws_sorted, offs, m_ids, e_ids


_DISP_ROWS = N_ASSIGN // SC_WORKERS
_DISP_CHUNK = 8
_DISP_STEPS = _DISP_ROWS // _DISP_CHUNK


def _dispatch_body(tok_hbm, x_hbm, xs_hbm, idx_v, buf0_v, buf1_v, sem0, sem1):
    wid = lax.axis_index("s") * SC_CORES + lax.axis_index("c")
    base = wid * _DISP_ROWS
    pltpu.sync_copy(tok_hbm.at[pl.ds(base, _DISP_ROWS)], idx_v)

    def _gather(c, buf, sem):
        iv = idx_v.at[pl.ds(c * _DISP_CHUNK, _DISP_CHUNK)]
        return pltpu.async_copy(x_hbm.at[iv], buf, sem)

    def _store(c, buf):
        pltpu.sync_copy(buf, xs_hbm.at[pl.ds(base + c * _DISP_CHUNK,
                                             _DISP_CHUNK)])

    _gather(0, buf0_v, sem0)

    @pl.loop(0, _DISP_STEPS // 2)
    def _pair(k):
        c0 = 2 * k
        _gather(c0 + 1, buf1_v, sem1)
        pltpu.make_async_copy(x_hbm.at[pl.ds(0, _DISP_CHUNK)],
                              buf0_v, sem0).wait()
        _store(c0, buf0_v)

        @pl.when(k < _DISP_STEPS // 2 - 1)
        def _():
            _gather(c0 + 2, buf0_v, sem0)

        pltpu.make_async_copy(x_hbm.at[pl.ds(0, _DISP_CHUNK)],
                              buf1_v, sem1).wait()
        _store(c0 + 1, buf1_v)


def _dispatch(tok_sorted, x):
    mesh = plsc.VectorSubcoreMesh(core_axis_name="c", subcore_axis_name="s")
    return pl.kernel(
        _dispatch_body,
        out_type=jax.ShapeDtypeStruct((N_ASSIGN, D_IN), jnp.float32),
        mesh=mesh,
        scratch_types=[
            pltpu.VMEM((_DISP_ROWS,), jnp.int32),
            pltpu.VMEM((_DISP_CHUNK, D_IN), jnp.float32),
            pltpu.VMEM((_DISP_CHUNK, D_IN), jnp.float32),
            pltpu.SemaphoreType.DMA,
            pltpu.SemaphoreType.DMA,
        ],
    )(tok_sorted, x)


def _gmm_body(m_ids, e_ids, offs, x_ref, w_ref, b_ref, ws_ref, o_ref):
    w = pl.program_id(1)
    e = e_ids[w]
    row0 = m_ids[w] * M_TILE
    lo = offs[e] - row0
    hi = offs[e + 1] - row0
    a = x_ref[...].astype(jnp.bfloat16)
    acc = jax.lax.dot_general(a, w_ref[0], (((1,), (0,)), ((), ())),
                              preferred_element_type=jnp.float32)
    acc = (acc + b_ref[0]) * ws_ref[...]
    rows = jax.lax.broadcasted_iota(jnp.int32, (M_TILE, N_TILE), 0)
    keep = (rows >= lo) & (rows < hi)
    o_ref[...] = jnp.where(keep, acc, o_ref[...])


def _gmm(m_ids, e_ids, offs, x_sorted, w_bf16, b, ws_sorted):
    grid_spec = pltpu.PrefetchScalarGridSpec(
        num_scalar_prefetch=3,
        grid=(N_TILES, N_WORKS),
        in_specs=[
            pl.BlockSpec((M_TILE, D_IN), lambda n, w, mi, ei, of: (mi[w], 0)),
            pl.BlockSpec((1, D_IN, N_TILE),
                         lambda n, w, mi, ei, of: (ei[w], 0, n)),
            pl.BlockSpec((1, 1, N_TILE), lambda n, w, mi, ei, of: (ei[w], 0, n)),
            pl.BlockSpec((M_TILE, 1), lambda n, w, mi, ei, of: (mi[w], 0)),
        ],
        out_specs=pl.BlockSpec((M_TILE, N_TILE),
                               lambda n, w, mi, ei, of: (mi[w], n)),
    )
    return pl.pallas_call(
        _gmm_body,
        grid_spec=grid_spec,
        out_shape=jax.ShapeDtypeStruct((N_ASSIGN, D_OUT), jnp.float32),
        compiler_params=pltpu.CompilerParams(
            vmem_limit_bytes=67000000),
    )(m_ids, e_ids, offs, x_sorted, w_bf16, b.reshape(N_EXP, 1, D_OUT),
      ws_sorted)


_CMB_TOK = N_TOK // SC_WORKERS
_CMB_CHUNK = 4
_CMB_STEPS = _CMB_TOK // _CMB_CHUNK


def _combine_body(p0_hbm, p1_hbm, y_hbm, out_hbm,
                  p0_v, p1_v, y0a_v, y1a_v, y0b_v, y1b_v, sema, semb):
    wid = lax.axis_index("s") * SC_CORES + lax.axis_index("c")
    base = wid * _CMB_TOK
    pltpu.sync_copy(p0_hbm.at[wid], p0_v)
    pltpu.sync_copy(p1_hbm.at[wid], p1_v)

    def _gathers(c, y0, y1, sem):
        pltpu.async_copy(y_hbm.at[p0_v.at[c]], y0, sem)
        pltpu.async_copy(y_hbm.at[p1_v.at[c]], y1, sem)

    def _drain(y0, y1, sem):
        pltpu.make_async_copy(y_hbm.at[pl.ds(0, _CMB_CHUNK)], y0, sem).wait()
        pltpu.make_async_copy(y_hbm.at[pl.ds(0, _CMB_CHUNK)], y1, sem).wait()

    def _add_store(c, y0, y1):
        for r in range(_CMB_CHUNK):

            @pl.loop(0, D_OUT // 16, unroll=8)
            def _col(j):
                sl = pl.ds(j * 16, 16)
                y0[r, sl] = y0[r, sl] + y1[r, sl]

        pltpu.sync_copy(y0, out_hbm.at[pl.ds(base + c * _CMB_CHUNK,
                                             _CMB_CHUNK)])

    _gathers(0, y0a_v, y1a_v, sema)

    @pl.loop(0, _CMB_STEPS // 2)
    def _pair(k):
        c0 = 2 * k
        _gathers(c0 + 1, y0b_v, y1b_v, semb)
        _drain(y0a_v, y1a_v, sema)
        _add_store(c0, y0a_v, y1a_v)

        @pl.when(k < _CMB_STEPS // 2 - 1)
        def _():
            _gathers(c0 + 2, y0a_v, y1a_v, sema)

        _drain(y0b_v, y1b_v, semb)
        _add_store(c0 + 1, y0b_v, y1b_v)


def _combine(p0, p1, y_sorted):
    mesh = plsc.VectorSubcoreMesh(core_axis_name="c", subcore_axis_name="s")
    return pl.kernel(
        _combine_body,
        out_type=jax.ShapeDtypeStruct((N_TOK, D_OUT), jnp.float32),
        mesh=mesh,
        scratch_types=[
            pltpu.VMEM((_CMB_STEPS, _CMB_CHUNK), jnp.int32),
            pltpu.VMEM((_CMB_STEPS, _CMB_CHUNK), jnp.int32),
            pltpu.VMEM((_CMB_CHUNK, D_OUT), jnp.float32),
            pltpu.VMEM((_CMB_CHUNK, D_OUT), jnp.float32),
            pltpu.VMEM((_CMB_CHUNK, D_OUT), jnp.float32),
            pltpu.VMEM((_CMB_CHUNK, D_OUT), jnp.float32),
            pltpu.SemaphoreType.DMA,
            pltpu.SemaphoreType.DMA,
        ],
    )(p0.reshape(SC_WORKERS, _CMB_STEPS, _CMB_CHUNK),
      p1.reshape(SC_WORKERS, _CMB_STEPS, _CMB_CHUNK), y_sorted)


def kernel(x, Wg, bg, W, b):
    wg_pad = jnp.pad(Wg, ((0, 0), (0, 128 - N_EXP)))
    bg_pad = jnp.pad(bg, (0, 128 - N_EXP)).reshape(1, 128)
    i0, i1, w0, w1 = _gating(x, wg_pad, bg_pad)
    i0, i1 = i0[:, 0], i1[:, 0]
    w0, w1 = w0[:, 0], w1[:, 0]
    tok_sorted, p0, p1, ws_sorted, offs, m_ids, e_ids = _routing_tables(
        i0, i1, w0, w1)
    x_sorted = _dispatch(tok_sorted, x)
    y_sorted = _gmm(m_ids, e_ids, offs, x_sorted, W.astype(jnp.bfloat16), b,
                    ws_sorted)
    return _combine(p0, p1, y_sorted)


# speedup vs baseline: 1.0205x; 1.0205x over previous
import functools

import jax
import jax.numpy as jnp
from jax import lax
from jax.experimental import pallas as pl
from jax.experimental.pallas import tpu as pltpu
from jax.experimental.pallas import tpu_sc as plsc

N_TOK = 8192
D_IN = 4096
D_OUT = 4096
N_EXP = 8
N_ASSIGN = 2 * N_TOK

M_TILE = 512
M_TILES = N_ASSIGN // M_TILE
N_WORKS = M_TILES + N_EXP - 1
N_TILE = 2048
N_TILES = D_OUT // N_TILE

SC_CORES = 2
SC_SUBCORES = 16
SC_WORKERS = SC_CORES * SC_SUBCORES

GATE_BM = 512


def _gating_body(x_ref, wg_ref, bg_ref, i0_ref, i1_ref, w0_ref, w1_ref):
    g = jax.lax.dot_general(
        x_ref[...], wg_ref[...], (((1,), (0,)), ((), ())),
        preferred_element_type=jnp.float32)
    g = g + bg_ref[...]
    lanes = jax.lax.broadcasted_iota(jnp.int32, (GATE_BM, 128), 1)
    real = lanes < N_EXP
    neg = jnp.where(real, g, -jnp.inf)
    m0 = jnp.max(neg, axis=1, keepdims=True)
    i0 = jnp.max(jnp.where((neg == m0) & real, lanes, -1), axis=1, keepdims=True)
    neg2 = jnp.where(lanes == i0, -jnp.inf, neg)
    m1 = jnp.max(neg2, axis=1, keepdims=True)
    i1 = jnp.max(jnp.where((neg2 == m1) & real, lanes, -1), axis=1, keepdims=True)
    e1 = jnp.exp(m1 - m0)
    denom = 1.0 + e1
    i0_ref[...] = i0
    i1_ref[...] = i1
    w0_ref[...] = 1.0 / denom
    w1_ref[...] = e1 / denom


def _gating(x, wg_pad, bg_pad):
    grid = (N_TOK // GATE_BM,)
    out1 = jax.ShapeDtypeStruct((N_TOK, 1), jnp.int32)
    outf = jax.ShapeDtypeStruct((N_TOK, 1), jnp.float32)
    return pl.pallas_call(
        _gating_body,
        grid=grid,
        in_specs=[
            pl.BlockSpec((GATE_BM, D_IN), lambda m: (m, 0)),
            pl.BlockSpec((D_IN, 128), lambda m: (0, 0)),
            pl.BlockSpec((1, 128), lambda m: (0, 0)),
        ],
        out_specs=[
            pl.BlockSpec((GATE_BM, 1), lambda m: (m, 0)),
            pl.BlockSpec((GATE_BM, 1), lambda m: (m, 0)),
            pl.BlockSpec((GATE_BM, 1), lambda m: (m, 0)),
            pl.BlockSpec((GATE_BM, 1), lambda m: (m, 0)),
        ],
        out_shape=[out1, out1, outf, outf],
    )(x, wg_pad, bg_pad)


def _routing_tables(i0, i1, w0, w1):
    e_flat = jnp.stack([i0, i1], axis=1).reshape(-1).astype(jnp.int32)
    onehot = (e_flat[:, None] == jnp.arange(N_EXP, dtype=jnp.int32)[None, :])
    ends = jnp.cumsum(onehot.astype(jnp.int32), axis=0)
    counts = ends[-1]
    offs = jnp.concatenate(
        [jnp.zeros((1,), jnp.int32), jnp.cumsum(counts)]).astype(jnp.int32)
    rank = jnp.take_along_axis(ends, e_flat[:, None], axis=1)[:, 0] - 1
    pos = (offs[e_flat] + rank).astype(jnp.int32)
    p0 = pos[0::2]
    p1 = pos[1::2]
    w_flat = jnp.stack([w0, w1], axis=1).reshape(-1)
    wbits = lax.bitcast_convert_type(w_flat, jnp.int32)
    packed = jnp.zeros((N_ASSIGN, 2), jnp.int32).at[pos].set(
        jnp.stack([jnp.arange(N_ASSIGN, dtype=jnp.int32) // 2, wbits],
                  axis=1))
    tok_sorted = packed[:, 0]
    ws_sorted = lax.bitcast_convert_type(packed[:, 1:2], jnp.float32)

    first_tile = offs[:N_EXP] // M_TILE
    last_tile = (offs[1:] - 1) // M_TILE
    ntiles = jnp.where(counts > 0, last_tile - first_tile + 1, 0)
    wend = jnp.cumsum(ntiles)
    wstart = jnp.concatenate([jnp.zeros((1,), wend.dtype), wend[:-1]])
    total = wend[N_EXP - 1]
    w_ar = jnp.arange(N_WORKS, dtype=jnp.int32)
    ew = jnp.minimum(jnp.searchsorted(wend, w_ar, side="right"), N_EXP - 1)
    mw = first_tile[ew] + (w_ar - wstart[ew])
    el = jnp.minimum(jnp.searchsorted(wend, total - 1, side="right"), N_EXP - 1)
    ml = first_tile[el] + (total - 1 - wstart[el])
    valid = w_ar < total
    e_ids = jnp.where(valid, ew, el).astype(jnp.int32)
    m_ids = jnp.where(valid, mw, ml).astype(jnp.int32)
    return tok_sorted, p0, p1, ws_sorted, offs, m_ids, e_ids


_DISP_ROWS = N_ASSIGN // SC_WORKERS
_DISP_CHUNK = 8
_DISP_STEPS = _DISP_ROWS // _DISP_CHUNK


def _dispatch_body(tok_hbm, x_hbm, xs_hbm, idx_v, buf0_v, buf1_v, sem0, sem1):
    wid = lax.axis_index("s") * SC_CORES + lax.axis_index("c")
    base = wid * _DISP_ROWS
    pltpu.sync_copy(tok_hbm.at[pl.ds(base, _DISP_ROWS)], idx_v)

    def _gather(c, buf, sem):
        iv = idx_v.at[pl.ds(c * _DISP_CHUNK, _DISP_CHUNK)]
        return pltpu.async_copy(x_hbm.at[iv], buf, sem)

    def _store(c, buf):
        pltpu.sync_copy(buf, xs_hbm.at[pl.ds(base + c * _DISP_CHUNK,
                                             _DISP_CHUNK)])

    _gather(0, buf0_v, sem0)

    @pl.loop(0, _DISP_STEPS // 2)
    def _pair(k):
        c0 = 2 * k
        _gather(c0 + 1, buf1_v, sem1)
        pltpu.make_async_copy(x_hbm.at[pl.ds(0, _DISP_CHUNK)],
                              buf0_v, sem0).wait()
        _store(c0, buf0_v)

        @pl.when(k < _DISP_STEPS // 2 - 1)
        def _():
            _gather(c0 + 2, buf0_v, sem0)

        pltpu.make_async_copy(x_hbm.at[pl.ds(0, _DISP_CHUNK)],
                              buf1_v, sem1).wait()
        _store(c0 + 1, buf1_v)


def _dispatch(tok_sorted, x):
    mesh = plsc.VectorSubcoreMesh(core_axis_name="c", subcore_axis_name="s")
    return pl.kernel(
        _dispatch_body,
        out_type=jax.ShapeDtypeStruct((N_ASSIGN, D_IN), jnp.float32),
        mesh=mesh,
        scratch_types=[
            pltpu.VMEM((_DISP_ROWS,), jnp.int32),
            pltpu.VMEM((_DISP_CHUNK, D_IN), jnp.float32),
            pltpu.VMEM((_DISP_CHUNK, D_IN), jnp.float32),
            pltpu.SemaphoreType.DMA,
            pltpu.SemaphoreType.DMA,
        ],
    )(tok_sorted, x)


def _gmm_body(m_ids, e_ids, offs, x_ref, w_ref, b_ref, ws_ref, o_ref):
    w = pl.program_id(1)
    e = e_ids[w]
    row0 = m_ids[w] * M_TILE
    lo = offs[e] - row0
    hi = offs[e + 1] - row0
    a = x_ref[...].astype(jnp.bfloat16)
    acc = jax.lax.dot_general(a, w_ref[0], (((1,), (0,)), ((), ())),
                              preferred_element_type=jnp.float32)
    acc = (acc + b_ref[0]) * ws_ref[...]
    rows = jax.lax.broadcasted_iota(jnp.int32, (M_TILE, N_TILE), 0)
    keep = (rows >= lo) & (rows < hi)
    o_ref[...] = jnp.where(keep, acc, o_ref[...])


def _gmm(m_ids, e_ids, offs, x_sorted, w_bf16, b, ws_sorted):
    grid_spec = pltpu.PrefetchScalarGridSpec(
        num_scalar_prefetch=3,
        grid=(N_TILES, N_WORKS),
        in_specs=[
            pl.BlockSpec((M_TILE, D_IN), lambda n, w, mi, ei, of: (mi[w], 0)),
            pl.BlockSpec((1, D_IN, N_TILE),
                         lambda n, w, mi, ei, of: (ei[w], 0, n)),
            pl.BlockSpec((1, 1, N_TILE), lambda n, w, mi, ei, of: (ei[w], 0, n)),
            pl.BlockSpec((M_TILE, 1), lambda n, w, mi, ei, of: (mi[w], 0)),
        ],
        out_specs=pl.BlockSpec((M_TILE, N_TILE),
                               lambda n, w, mi, ei, of: (mi[w], n)),
    )
    return pl.pallas_call(
        _gmm_body,
        grid_spec=grid_spec,
        out_shape=jax.ShapeDtypeStruct((N_ASSIGN, D_OUT), jnp.float32),
        compiler_params=pltpu.CompilerParams(
            vmem_limit_bytes=67000000),
    )(m_ids, e_ids, offs, x_sorted, w_bf16, b.reshape(N_EXP, 1, D_OUT),
      ws_sorted)


_CMB_TOK = N_TOK // SC_WORKERS
_CMB_CHUNK = 4
_CMB_STEPS = _CMB_TOK // _CMB_CHUNK


def _combine_body(p0_hbm, p1_hbm, y_hbm, out_hbm,
                  p0_v, p1_v, y0a_v, y1a_v, y0b_v, y1b_v, sema, semb):
    wid = lax.axis_index("s") * SC_CORES + lax.axis_index("c")
    base = wid * _CMB_TOK
    pltpu.sync_copy(p0_hbm.at[wid], p0_v)
    pltpu.sync_copy(p1_hbm.at[wid], p1_v)

    def _gathers(c, y0, y1, sem):
        pltpu.async_copy(y_hbm.at[p0_v.at[c]], y0, sem)
        pltpu.async_copy(y_hbm.at[p1_v.at[c]], y1, sem)

    def _drain(y0, y1, sem):
        pltpu.make_async_copy(y_hbm.at[pl.ds(0, _CMB_CHUNK)], y0, sem).wait()
        pltpu.make_async_copy(y_hbm.at[pl.ds(0, _CMB_CHUNK)], y1, sem).wait()

    def _add_store(c, y0, y1):
        for r in range(_CMB_CHUNK):

            @pl.loop(0, D_OUT // 16, unroll=8)
            def _col(j):
                sl = pl.ds(j * 16, 16)
                y0[r, sl] = y0[r, sl] + y1[r, sl]

        pltpu.sync_copy(y0, out_hbm.at[pl.ds(base + c * _CMB_CHUNK,
                                             _CMB_CHUNK)])

    _gathers(0, y0a_v, y1a_v, sema)

    @pl.loop(0, _CMB_STEPS // 2)
    def _pair(k):
        c0 = 2 * k
        _gathers(c0 + 1, y0b_v, y1b_v, semb)
        _drain(y0a_v, y1a_v, sema)
        _add_store(c0, y0a_v, y1a_v)

        @pl.when(k < _CMB_STEPS // 2 - 1)
        def _():
            _gathers(c0 + 2, y0a_v, y1a_v, sema)

        _drain(y0b_v, y1b_v, semb)
        _add_store(c0 + 1, y0b_v, y1b_v)


def _combine(p0, p1, y_sorted):
    mesh = plsc.VectorSubcoreMesh(core_axis_name="c", subcore_axis_name="s")
    return pl.kernel(
        _combine_body,
        out_type=jax.ShapeDtypeStruct((N_TOK, D_OUT), jnp.float32),
        mesh=mesh,
        scratch_types=[
            pltpu.VMEM((_CMB_STEPS, _CMB_CHUNK), jnp.int32),
            pltpu.VMEM((_CMB_STEPS, _CMB_CHUNK), jnp.int32),
            pltpu.VMEM((_CMB_CHUNK, D_OUT), jnp.float32),
            pltpu.VMEM((_CMB_CHUNK, D_OUT), jnp.float32),
            pltpu.VMEM((_CMB_CHUNK, D_OUT), jnp.float32),
            pltpu.VMEM((_CMB_CHUNK, D_OUT), jnp.float32),
            pltpu.SemaphoreType.DMA,
            pltpu.SemaphoreType.DMA,
        ],
    )(p0.reshape(SC_WORKERS, _CMB_STEPS, _CMB_CHUNK),
      p1.reshape(SC_WORKERS, _CMB_STEPS, _CMB_CHUNK), y_sorted)


def kernel(x, Wg, bg, W, b):
    wg_pad = jnp.pad(Wg, ((0, 0), (0, 128 - N_EXP)))
    bg_pad = jnp.pad(bg, (0, 128 - N_EXP)).reshape(1, 128)
    i0, i1, w0, w1 = _gating(x, wg_pad, bg_pad)
    i0, i1 = i0[:, 0], i1[:, 0]
    w0, w1 = w0[:, 0], w1[:, 0]
    tok_sorted, p0, p1, ws_sorted, offs, m_ids, e_ids = _routing_tables(
        i0, i1, w0, w1)
    x_sorted = _dispatch(tok_sorted, x)
    y_sorted = _gmm(m_ids, e_ids, offs, x_sorted, W.astype(jnp.bfloat16), b,
                    ws_sorted)
    return _combine(p0, p1, y_sorted)


# speedup vs baseline: 1.0206x; 1.0002x over previous
import jax
import jax.numpy as jnp
from jax import lax
from jax.experimental import pallas as pl
from jax.experimental.pallas import tpu as pltpu
from jax.experimental.pallas import tpu_sc as plsc

N_TOK = 8192
D_IN = 4096
D_OUT = 4096
N_EXP = 8
N_ASSIGN = 2 * N_TOK

M_TILE = 512
M_TILES = N_ASSIGN // M_TILE
N_WORKS = M_TILES + N_EXP - 1
N_TILE = 2048
N_TILES = D_OUT // N_TILE

SC_CORES = 2
SC_SUBCORES = 16
SC_WORKERS = SC_CORES * SC_SUBCORES

GATE_BM = 512


def _gating_body(x_ref, wg_ref, bg_ref, i0_ref, i1_ref, w0_ref, w1_ref):
    g = jax.lax.dot_general(
        x_ref[...], wg_ref[...], (((1,), (0,)), ((), ())),
        preferred_element_type=jnp.float32)
    g = g + bg_ref[...]
    lanes = jax.lax.broadcasted_iota(jnp.int32, (GATE_BM, 128), 1)
    real = lanes < N_EXP
    neg = jnp.where(real, g, -jnp.inf)
    m0 = jnp.max(neg, axis=1, keepdims=True)
    i0 = jnp.max(jnp.where((neg == m0) & real, lanes, -1), axis=1, keepdims=True)
    neg2 = jnp.where(lanes == i0, -jnp.inf, neg)
    m1 = jnp.max(neg2, axis=1, keepdims=True)
    i1 = jnp.max(jnp.where((neg2 == m1) & real, lanes, -1), axis=1, keepdims=True)
    e1 = jnp.exp(m1 - m0)
    denom = 1.0 + e1
    i0_ref[...] = i0
    i1_ref[...] = i1
    w0_ref[...] = 1.0 / denom
    w1_ref[...] = e1 / denom


def _gating(x, wg_pad, bg_pad):
    grid = (N_TOK // GATE_BM,)
    out1 = jax.ShapeDtypeStruct((N_TOK, 1), jnp.int32)
    outf = jax.ShapeDtypeStruct((N_TOK, 1), jnp.float32)
    return pl.pallas_call(
        _gating_body,
        grid=grid,
        in_specs=[
            pl.BlockSpec((GATE_BM, D_IN), lambda m: (m, 0)),
            pl.BlockSpec((D_IN, 128), lambda m: (0, 0)),
            pl.BlockSpec((1, 128), lambda m: (0, 0)),
        ],
        out_specs=[
            pl.BlockSpec((GATE_BM, 1), lambda m: (m, 0)),
            pl.BlockSpec((GATE_BM, 1), lambda m: (m, 0)),
            pl.BlockSpec((GATE_BM, 1), lambda m: (m, 0)),
            pl.BlockSpec((GATE_BM, 1), lambda m: (m, 0)),
        ],
        out_shape=[out1, out1, outf, outf],
    )(x, wg_pad, bg_pad)


def _routing_tables(i0, i1, w0, w1):
    e_flat = jnp.stack([i0, i1], axis=1).reshape(-1).astype(jnp.int32)
    onehot = (e_flat[:, None] == jnp.arange(N_EXP, dtype=jnp.int32)[None, :])
    ends = jnp.cumsum(onehot.astype(jnp.int32), axis=0)
    counts = ends[-1]
    offs = jnp.concatenate(
        [jnp.zeros((1,), jnp.int32), jnp.cumsum(counts)]).astype(jnp.int32)
    rank = jnp.take_along_axis(ends, e_flat[:, None], axis=1)[:, 0] - 1
    pos = (offs[e_flat] + rank).astype(jnp.int32)
    p0 = pos[0::2]
    p1 = pos[1::2]
    w_flat = jnp.stack([w0, w1], axis=1).reshape(-1)
    wbits = lax.bitcast_convert_type(w_flat, jnp.int32)
    packed = jnp.zeros((N_ASSIGN, 2), jnp.int32).at[pos].set(
        jnp.stack([jnp.arange(N_ASSIGN, dtype=jnp.int32) // 2, wbits],
                  axis=1))
    tok_sorted = packed[:, 0]
    ws_sorted = lax.bitcast_convert_type(packed[:, 1:2], jnp.float32)

    first_tile = offs[:N_EXP] // M_TILE
    last_tile = (offs[1:] - 1) // M_TILE
    ntiles = jnp.where(counts > 0, last_tile - first_tile + 1, 0)
    wend = jnp.cumsum(ntiles)
    wstart = jnp.concatenate([jnp.zeros((1,), wend.dtype), wend[:-1]])
    total = wend[N_EXP - 1]
    w_ar = jnp.arange(N_WORKS, dtype=jnp.int32)
    ew = jnp.minimum(jnp.searchsorted(wend, w_ar, side="right"), N_EXP - 1)
    mw = first_tile[ew] + (w_ar - wstart[ew])
    el = jnp.minimum(jnp.searchsorted(wend, total - 1, side="right"), N_EXP - 1)
    ml = first_tile[el] + (total - 1 - wstart[el])
    valid = w_ar < total
    e_ids = jnp.where(valid, ew, el).astype(jnp.int32)
    m_ids = jnp.where(valid, mw, ml).astype(jnp.int32)
    return tok_sorted, p0, p1, ws_sorted, offs, m_ids, e_ids


_DISP_ROWS = N_ASSIGN // SC_WORKERS
_DISP_CHUNK = 8
_DISP_STEPS = _DISP_ROWS // _DISP_CHUNK


def _dispatch_body(tok_hbm, x_hbm, xs_hbm, idx_v, buf0_v, buf1_v, sem0, sem1):
    wid = lax.axis_index("s") * SC_CORES + lax.axis_index("c")
    base = wid * _DISP_ROWS
    pltpu.sync_copy(tok_hbm.at[pl.ds(base, _DISP_ROWS)], idx_v)

    def _gather(c, buf, sem):
        iv = idx_v.at[pl.ds(c * _DISP_CHUNK, _DISP_CHUNK)]
        return pltpu.async_copy(x_hbm.at[iv], buf, sem)

    def _store(c, buf):
        pltpu.sync_copy(buf, xs_hbm.at[pl.ds(base + c * _DISP_CHUNK,
                                             _DISP_CHUNK)])

    _gather(0, buf0_v, sem0)

    @pl.loop(0, _DISP_STEPS // 2)
    def _pair(k):
        c0 = 2 * k
        _gather(c0 + 1, buf1_v, sem1)
        pltpu.make_async_copy(x_hbm.at[pl.ds(0, _DISP_CHUNK)],
                              buf0_v, sem0).wait()
        _store(c0, buf0_v)

        @pl.when(k < _DISP_STEPS // 2 - 1)
        def _():
            _gather(c0 + 2, buf0_v, sem0)

        pltpu.make_async_copy(x_hbm.at[pl.ds(0, _DISP_CHUNK)],
                              buf1_v, sem1).wait()
        _store(c0 + 1, buf1_v)


def _dispatch(tok_sorted, x):
    mesh = plsc.VectorSubcoreMesh(core_axis_name="c", subcore_axis_name="s")
    return pl.kernel(
        _dispatch_body,
        out_type=jax.ShapeDtypeStruct((N_ASSIGN, D_IN), jnp.float32),
        mesh=mesh,
        scratch_types=[
            pltpu.VMEM((_DISP_ROWS,), jnp.int32),
            pltpu.VMEM((_DISP_CHUNK, D_IN), jnp.float32),
            pltpu.VMEM((_DISP_CHUNK, D_IN), jnp.float32),
            pltpu.SemaphoreType.DMA,
            pltpu.SemaphoreType.DMA,
        ],
    )(tok_sorted, x)


def _gmm_body(m_ids, e_ids, offs, x_ref, w_ref, b_ref, ws_ref, o_ref):
    w = pl.program_id(1)
    e = e_ids[w]
    row0 = m_ids[w] * M_TILE
    lo = offs[e] - row0
    hi = offs[e + 1] - row0
    a = x_ref[...].astype(jnp.bfloat16)
    acc = jax.lax.dot_general(a, w_ref[0], (((1,), (0,)), ((), ())),
                              preferred_element_type=jnp.float32)
    acc = (acc + b_ref[0]) * ws_ref[...]
    rows = jax.lax.broadcasted_iota(jnp.int32, (M_TILE, N_TILE), 0)
    keep = (rows >= lo) & (rows < hi)
    o_ref[...] = jnp.where(keep, acc, o_ref[...])


def _gmm(m_ids, e_ids, offs, x_sorted, w_bf16, b, ws_sorted):
    grid_spec = pltpu.PrefetchScalarGridSpec(
        num_scalar_prefetch=3,
        grid=(N_TILES, N_WORKS),
        in_specs=[
            pl.BlockSpec((M_TILE, D_IN), lambda n, w, mi, ei, of: (mi[w], 0)),
            pl.BlockSpec((1, D_IN, N_TILE),
                         lambda n, w, mi, ei, of: (ei[w], 0, n)),
            pl.BlockSpec((1, 1, N_TILE), lambda n, w, mi, ei, of: (ei[w], 0, n)),
            pl.BlockSpec((M_TILE, 1), lambda n, w, mi, ei, of: (mi[w], 0)),
        ],
        out_specs=pl.BlockSpec((M_TILE, N_TILE),
                               lambda n, w, mi, ei, of: (mi[w], n)),
    )
    return pl.pallas_call(
        _gmm_body,
        grid_spec=grid_spec,
        out_shape=jax.ShapeDtypeStruct((N_ASSIGN, D_OUT), jnp.float32),
        compiler_params=pltpu.CompilerParams(
            vmem_limit_bytes=67000000),
    )(m_ids, e_ids, offs, x_sorted, w_bf16, b.reshape(N_EXP, 1, D_OUT),
      ws_sorted)


_CMB_TOK = N_TOK // SC_WORKERS
_CMB_CHUNK = 4
_CMB_STEPS = _CMB_TOK // _CMB_CHUNK


def _combine_body(p0_hbm, p1_hbm, y_hbm, out_hbm,
                  p0_v, p1_v, y0a_v, y1a_v, y0b_v, y1b_v, sema, semb):
    wid = lax.axis_index("s") * SC_CORES + lax.axis_index("c")
    base = wid * _CMB_TOK
    pltpu.sync_copy(p0_hbm.at[wid], p0_v)
    pltpu.sync_copy(p1_hbm.at[wid], p1_v)

    def _gathers(c, y0, y1, sem):
        pltpu.async_copy(y_hbm.at[p0_v.at[c]], y0, sem)
        pltpu.async_copy(y_hbm.at[p1_v.at[c]], y1, sem)

    def _drain(y0, y1, sem):
        pltpu.make_async_copy(y_hbm.at[pl.ds(0, _CMB_CHUNK)], y0, sem).wait()
        pltpu.make_async_copy(y_hbm.at[pl.ds(0, _CMB_CHUNK)], y1, sem).wait()

    def _add_store(c, y0, y1):
        for r in range(_CMB_CHUNK):

            @pl.loop(0, D_OUT // 16, unroll=8)
            def _col(j):
                sl = pl.ds(j * 16, 16)
                y0[r, sl] = y0[r, sl] + y1[r, sl]

        pltpu.sync_copy(y0, out_hbm.at[pl.ds(base + c * _CMB_CHUNK,
                                             _CMB_CHUNK)])

    _gathers(0, y0a_v, y1a_v, sema)

    @pl.loop(0, _CMB_STEPS // 2)
    def _pair(k):
        c0 = 2 * k
        _gathers(c0 + 1, y0b_v, y1b_v, semb)
        _drain(y0a_v, y1a_v, sema)
        _add_store(c0, y0a_v, y1a_v)

        @pl.when(k < _CMB_STEPS // 2 - 1)
        def _():
            _gathers(c0 + 2, y0a_v, y1a_v, sema)

        _drain(y0b_v, y1b_v, semb)
        _add_store(c0 + 1, y0b_v, y1b_v)


def _combine(p0, p1, y_sorted):
    mesh = plsc.VectorSubcoreMesh(core_axis_name="c", subcore_axis_name="s")
    return pl.kernel(
        _combine_body,
        out_type=jax.ShapeDtypeStruct((N_TOK, D_OUT), jnp.float32),
        mesh=mesh,
        scratch_types=[
            pltpu.VMEM((_CMB_STEPS, _CMB_CHUNK), jnp.int32),
            pltpu.VMEM((_CMB_STEPS, _CMB_CHUNK), jnp.int32),
            pltpu.VMEM((_CMB_CHUNK, D_OUT), jnp.float32),
            pltpu.VMEM((_CMB_CHUNK, D_OUT), jnp.float32),
            pltpu.VMEM((_CMB_CHUNK, D_OUT), jnp.float32),
            pltpu.VMEM((_CMB_CHUNK, D_OUT), jnp.float32),
            pltpu.SemaphoreType.DMA,
            pltpu.SemaphoreType.DMA,
        ],
    )(p0.reshape(SC_WORKERS, _CMB_STEPS, _CMB_CHUNK),
      p1.reshape(SC_WORKERS, _CMB_STEPS, _CMB_CHUNK), y_sorted)


def kernel(x, Wg, bg, W, b):
    wg_pad = jnp.pad(Wg, ((0, 0), (0, 128 - N_EXP)))
    bg_pad = jnp.pad(bg, (0, 128 - N_EXP)).reshape(1, 128)
    i0, i1, w0, w1 = _gating(x, wg_pad, bg_pad)
    i0, i1 = i0[:, 0], i1[:, 0]
    w0, w1 = w0[:, 0], w1[:, 0]
    tok_sorted, p0, p1, ws_sorted, offs, m_ids, e_ids = _routing_tables(
        i0, i1, w0, w1)
    x_sorted = _dispatch(tok_sorted, x)
    y_sorted = _gmm(m_ids, e_ids, offs, x_sorted, W.astype(jnp.bfloat16), b,
                    ws_sorted)
    return _combine(p0, p1, y_sorted)
